# Initial kernel scaffold; baseline (speedup 1.0000x reference)
import jax
import jax.numpy as jnp
from jax.experimental import pallas as pl


def kernel(x, edge_index, batch, bn_gamma, bn_beta, W1, b1, W2, b2, W3, b3, lin_W, lin_b):
    raise NotImplementedError("write your pallas kernel here")


# speedup vs baseline: 3.8681x; 3.8681x over previous
import functools
import jax
import jax.numpy as jnp
from jax import lax
from jax.experimental import pallas as pl
from jax.experimental.pallas import tpu as pltpu

N = 10000
E = 160000
D = 256
H = 512
C = 40
G = 64

BLK = 1000
NBLK = N // BLK


def _bn_body(x_ref, gamma_ref, beta_ref, dis_ref, o_ref):
    x = x_ref[...]
    mean = jnp.mean(x, axis=0, keepdims=True)
    var = jnp.mean((x - mean) ** 2, axis=0, keepdims=True)
    xh = (x - mean) * jax.lax.rsqrt(var + 1e-5) * gamma_ref[...] + beta_ref[...]
    o_ref[...] = xh * dis_ref[...]


def _bn_call(x, gamma, beta, dis):
    return pl.pallas_call(
        _bn_body,
        out_shape=jax.ShapeDtypeStruct((N, D), jnp.float32),
    )(x, gamma.reshape(1, D), beta.reshape(1, D), dis.reshape(N, 1))


def _mm_scale_body(acc_ref, dis_ref, w_ref, b_ref, o_ref):
    a = acc_ref[...] * dis_ref[...]
    h = jnp.dot(a, w_ref[...], preferred_element_type=jnp.float32) + b_ref[...]
    o_ref[...] = jnp.maximum(h, 0.0) * dis_ref[...]


def _mm_scale_call(acc, dis, w, b, din, dout):
    return pl.pallas_call(
        _mm_scale_body,
        grid=(NBLK,),
        in_specs=[
            pl.BlockSpec((BLK, din), lambda i: (i, 0)),
            pl.BlockSpec((BLK, 1), lambda i: (i, 0)),
            pl.BlockSpec((din, dout), lambda i: (0, 0)),
            pl.BlockSpec((1, dout), lambda i: (0, 0)),
        ],
        out_specs=pl.BlockSpec((BLK, dout), lambda i: (i, 0)),
        out_shape=jax.ShapeDtypeStruct((N, dout), jnp.float32),
    )(acc, dis.reshape(N, 1), w, b.reshape(1, dout))


def _mm_pool_body(acc_ref, dis_ref, w2_ref, b2_ref, wt_ref, o_ref):
    i = pl.program_id(0)
    a = acc_ref[...] * dis_ref[...]
    h2 = jnp.dot(a, w2_ref[...], preferred_element_type=jnp.float32) + b2_ref[...]
    h2 = jnp.maximum(h2, 0.0)
    rt = wt_ref[...] * dis_ref[...]
    contrib = lax.dot_general(rt, h2, (((0,), (0,)), ((), ())),
                              preferred_element_type=jnp.float32)

    @pl.when(i == 0)
    def _():
        o_ref[...] = jnp.zeros_like(o_ref)

    o_ref[...] += contrib


def _mm_pool_call(acc2, dis, w2, b2, wt):
    return pl.pallas_call(
        _mm_pool_body,
        grid=(NBLK,),
        in_specs=[
            pl.BlockSpec((BLK, H), lambda i: (i, 0)),
            pl.BlockSpec((BLK, 1), lambda i: (i, 0)),
            pl.BlockSpec((H, H), lambda i: (0, 0)),
            pl.BlockSpec((1, H), lambda i: (0, 0)),
            pl.BlockSpec((BLK, G), lambda i: (i, 0)),
        ],
        out_specs=pl.BlockSpec((G, H), lambda i: (0, 0)),
        out_shape=jax.ShapeDtypeStruct((G, H), jnp.float32),
    )(acc2, dis.reshape(N, 1), w2, b2.reshape(1, H), wt)


def _final_body(pp_ref, w3_ref, b3_ref, cp_ref, lw_ref, lb_ref, o_ref):
    pooled = jnp.dot(pp_ref[...], w3_ref[...],
                     preferred_element_type=jnp.float32)
    pooled = pooled + b3_ref[...] * cp_ref[...]
    o_ref[...] = jnp.dot(pooled, lw_ref[...],
                         preferred_element_type=jnp.float32) + lb_ref[...]


def _final_call(pooled_pre, w3, b3, cntpos, lin_w, lin_b):
    return pl.pallas_call(
        _final_body,
        out_shape=jax.ShapeDtypeStruct((G, C), jnp.float32),
    )(pooled_pre, w3, b3.reshape(1, H), cntpos.reshape(G, 1),
      lin_w, lin_b.reshape(1, C))


def kernel(x, edge_index, batch, bn_gamma, bn_beta, W1, b1, W2, b2, W3, b3,
           lin_W, lin_b):
    src = edge_index[0]
    dst = edge_index[1]

    deg = jax.ops.segment_sum(jnp.ones((E,), jnp.float32), dst,
                              num_segments=N) + 1.0
    dis = deg ** -0.5

    def prop(zs):
        return zs + jax.ops.segment_sum(zs[src], dst, num_segments=N)

    zs1 = _bn_call(x, bn_gamma, bn_beta, dis)
    acc1 = prop(zs1)
    zs2 = _mm_scale_call(acc1, dis, W1, b1, D, H)
    acc2 = prop(zs2)

    onehot = (batch[:, None] == jnp.arange(G)[None, :]).astype(jnp.float32)
    cnt = jnp.sum(onehot, axis=0)
    u = onehot * (dis / jnp.maximum(cnt, 1.0)[batch])[:, None]
    w = u + jax.ops.segment_sum(u[dst], src, num_segments=N)

    pooled_pre = _mm_pool_call(acc2, dis, W2, b2, w)
    cntpos = (cnt > 0).astype(jnp.float32)
    return _final_call(pooled_pre, W3, b3, cntpos, lin_W, lin_b)


# speedup vs baseline: 7.2239x; 1.8676x over previous
import functools
import jax
import jax.numpy as jnp
from jax import lax
from jax.experimental import pallas as pl
from jax.experimental.pallas import tpu as pltpu
from jax.experimental.pallas import tpu_sc as plsc

N = 10000
E = 160000
D = 256
H = 512
C = 40
G = 64

BLK = 1000
NBLK = N // BLK

NC = 2
NS = 16
NW = NC * NS
CHUNK = 128
EPT = 5120
NCH = EPT // CHUNK
EPAD = NW * EPT
DUMMY = N
NACC = 10240
ZR = NACC // NS
WRA = 624
WRL = N - WRA * (NS - 1)


def _sc_mesh():
    return plsc.VectorSubcoreMesh(core_axis_name="c", subcore_axis_name="s")


def _zero_fill(ref, rows, width):
    def row(i, _):
        for t in range(width // 16):
            ref[i, pl.ds(t * 16, 16)] = jnp.zeros((16,), jnp.float32)
        return 0
    lax.fori_loop(0, rows, row, 0, unroll=False)


def _make_sc_prop(parts, width):
    scratch = [
        pltpu.VMEM((2, CHUNK, width), jnp.float32),
        pltpu.VMEM((NCH, CHUNK), jnp.int32),
        pltpu.VMEM((NCH, CHUNK), jnp.int32),
        pltpu.VMEM((16, width), jnp.float32),
        pltpu.VMEM_SHARED((NACC, width), jnp.float32),
        pltpu.SemaphoreType.DMA,
        pltpu.SemaphoreType.DMA,
        pltpu.SemaphoreType.DMA,
        pltpu.SemaphoreType.DMA,
    ]

    @functools.partial(
        pl.kernel,
        out_type=jax.ShapeDtypeStruct((NC, parts, N, width), jnp.float32),
        mesh=_sc_mesh(),
        scratch_types=scratch,
    )
    def k(*refs):
        z_hbm = refs[:parts]
        gidx_hbm, sidx_hbm, out_hbm = refs[parts:parts + 3]
        bufs, gidx, sidx, zblk, acc, sg0, sg1, ss0, ss1 = refs[parts + 3:]
        c = lax.axis_index("c")
        s = lax.axis_index("s")
        wid = s * NC + c

        pltpu.sync_copy(gidx_hbm.at[wid], gidx)
        pltpu.sync_copy(sidx_hbm.at[wid], sidx)
        _zero_fill(zblk, 16, width)

        sgs = (sg0, sg1)
        sss = (ss0, ss1)

        for p in range(parts):
            zp = z_hbm[p]

            def zcp(i, _):
                pltpu.sync_copy(zblk, acc.at[pl.ds(s * ZR + i * 16, 16)])
                return 0
            lax.fori_loop(0, ZR // 16, zcp, 0, unroll=False)
            plsc.subcore_barrier()

            pltpu.async_copy(zp.at[gidx.at[0]], bufs.at[0], sg0)
            pltpu.async_copy(zp.at[gidx.at[1]], bufs.at[1], sg1)

            def step(j, _):
                for b in range(2):
                    jj = j + b
                    pltpu.make_async_copy(
                        zp.at[gidx.at[jj]], bufs.at[b], sgs[b]).wait()
                    pltpu.async_copy(
                        bufs.at[b], acc.at[sidx.at[jj]], sss[b],
                        add=True).wait()
                    pltpu.async_copy(
                        zp.at[gidx.at[jj + 2]], bufs.at[b], sgs[b])
                return 0
            lax.fori_loop(0, (NCH - 2) // 2, lambda i, cc: step(i * 2, cc),
                          0, unroll=False)
            for b in range(2):
                jj = NCH - 2 + b
                pltpu.make_async_copy(
                    zp.at[gidx.at[jj]], bufs.at[b], sgs[b]).wait()
                pltpu.async_copy(
                    bufs.at[b], acc.at[sidx.at[jj]], sss[b], add=True).wait()

            plsc.subcore_barrier()

            @pl.when(s < NS - 1)
            def _():
                pltpu.sync_copy(acc.at[pl.ds(s * WRA, WRA)],
                                out_hbm.at[c, p, pl.ds(s * WRA, WRA)])

            @pl.when(s == NS - 1)
            def _():
                pltpu.sync_copy(acc.at[pl.ds(WRA * (NS - 1), WRL)],
                                out_hbm.at[c, p, pl.ds(WRA * (NS - 1), WRL)])

            plsc.subcore_barrier()

    return k


def _make_sc_degree():
    W16 = 128
    scratch = [
        pltpu.VMEM((CHUNK, W16), jnp.float32),
        pltpu.VMEM((NCH, CHUNK), jnp.int32),
        pltpu.VMEM((16, W16), jnp.float32),
        pltpu.VMEM_SHARED((NACC, W16), jnp.float32),
        pltpu.SemaphoreType.DMA,
        pltpu.SemaphoreType.DMA,
    ]

    @functools.partial(
        pl.kernel,
        out_type=jax.ShapeDtypeStruct((NC, N, W16), jnp.float32),
        mesh=_sc_mesh(),
        scratch_types=scratch,
    )
    def k(sidx_hbm, out_hbm, ones, sidx, zblk, acc, ss0, ss1):
        c = lax.axis_index("c")
        s = lax.axis_index("s")
        wid = s * NC + c

        pltpu.sync_copy(sidx_hbm.at[wid], sidx)
        _zero_fill(zblk, 16, W16)

        def ofill(i, _):
            for t in range(W16 // 16):
                ones[i, pl.ds(t * 16, 16)] = jnp.ones((16,), jnp.float32)
            return 0
        lax.fori_loop(0, CHUNK, ofill, 0, unroll=False)

        def zcp(i, _):
            pltpu.sync_copy(zblk, acc.at[pl.ds(s * ZR + i * 16, 16)])
            return 0
        lax.fori_loop(0, ZR // 16, zcp, 0, unroll=False)
        plsc.subcore_barrier()

        sss = (ss0, ss1)

        def step(j, _):
            for b in range(2):
                pltpu.async_copy(ones, acc.at[sidx.at[j + b]], sss[b],
                                 add=True)
            for b in range(2):
                pltpu.make_async_copy(ones, acc.at[sidx.at[j + b]],
                                      sss[b]).wait()
            return 0
        lax.fori_loop(0, NCH // 2, lambda i, cc: step(i * 2, cc), 0,
                      unroll=False)

        plsc.subcore_barrier()

        @pl.when(s < NS - 1)
        def _():
            pltpu.sync_copy(acc.at[pl.ds(s * WRA, WRA)],
                            out_hbm.at[c, pl.ds(s * WRA, WRA)])

        @pl.when(s == NS - 1)
        def _():
            pltpu.sync_copy(acc.at[pl.ds(WRA * (NS - 1), WRL)],
                            out_hbm.at[c, pl.ds(WRA * (NS - 1), WRL)])

        plsc.subcore_barrier()

    return k


def _stats_body(x_ref, batch_ref, stats_ref, cnt_ref):
    i = pl.program_id(0)
    x = x_ref[...]
    giota = lax.broadcasted_iota(jnp.int32, (1, G), 1)
    oh = (batch_ref[...] == giota).astype(jnp.float32)

    @pl.when(i == 0)
    def _():
        stats_ref[...] = jnp.zeros_like(stats_ref)
        cnt_ref[...] = jnp.zeros_like(cnt_ref)

    stats_ref[0:1] += jnp.sum(x, axis=0, keepdims=True)
    stats_ref[1:2] += jnp.sum(x * x, axis=0, keepdims=True)
    cnt_ref[...] += jnp.sum(oh, axis=0, keepdims=True)


def _stats_call(x, batch):
    return pl.pallas_call(
        _stats_body,
        grid=(NBLK,),
        in_specs=[
            pl.BlockSpec((BLK, D), lambda i: (i, 0)),
            pl.BlockSpec((BLK, 1), lambda i: (i, 0)),
        ],
        out_specs=(
            pl.BlockSpec((2, D), lambda i: (0, 0)),
            pl.BlockSpec((1, G), lambda i: (0, 0)),
        ),
        out_shape=(
            jax.ShapeDtypeStruct((2, D), jnp.float32),
            jax.ShapeDtypeStruct((1, G), jnp.float32),
        ),
    )(x, batch.reshape(N, 1))


def _bn_body(x_ref, gamma_ref, beta_ref, stats_ref, cnt_ref, deg_ref,
             batch_ref, zs_ref, u_ref, dis_ref):
    x = x_ref[...]
    mean = stats_ref[0:1] * (1.0 / N)
    var = stats_ref[1:2] * (1.0 / N) - mean * mean
    xh = (x - mean) * lax.rsqrt(var + 1e-5) * gamma_ref[...] + beta_ref[...]
    deg = deg_ref[0, :, 0:1] + deg_ref[1, :, 0:1] + 1.0
    dis = lax.rsqrt(deg)
    zs = xh * dis
    zs_ref[0] = zs[:, :128]
    zs_ref[1] = zs[:, 128:]
    dis_ref[...] = dis
    giota = lax.broadcasted_iota(jnp.int32, (1, G), 1)
    oh = (batch_ref[...] == giota).astype(jnp.float32)
    cnt = cnt_ref[...]
    denom = jnp.dot(oh, jnp.maximum(cnt, 1.0).reshape(G, 1),
                    preferred_element_type=jnp.float32)
    u_ref[...] = jnp.concatenate(
        [oh * (dis / denom), jnp.zeros((BLK, 128 - G), jnp.float32)],
        axis=-1)


def _bn_call(x, gamma, beta, stats, cnt, deg2, batch):
    return pl.pallas_call(
        _bn_body,
        grid=(NBLK,),
        in_specs=[
            pl.BlockSpec((BLK, D), lambda i: (i, 0)),
            pl.BlockSpec((1, D), lambda i: (0, 0)),
            pl.BlockSpec((1, D), lambda i: (0, 0)),
            pl.BlockSpec((2, D), lambda i: (0, 0)),
            pl.BlockSpec((1, G), lambda i: (0, 0)),
            pl.BlockSpec((2, BLK, 128), lambda i: (0, i, 0)),
            pl.BlockSpec((BLK, 1), lambda i: (i, 0)),
        ],
        out_specs=(
            pl.BlockSpec((2, BLK, 128), lambda i: (0, i, 0)),
            pl.BlockSpec((BLK, 128), lambda i: (i, 0)),
            pl.BlockSpec((BLK, 1), lambda i: (i, 0)),
        ),
        out_shape=(
            jax.ShapeDtypeStruct((2, N, 128), jnp.float32),
            jax.ShapeDtypeStruct((N, 128), jnp.float32),
            jax.ShapeDtypeStruct((N, 1), jnp.float32),
        ),
    )(x, gamma.reshape(1, D), beta.reshape(1, D), stats, cnt, deg2,
      batch.reshape(N, 1))


def _mm1_body(acc_ref, zs_ref, dis_ref, w_ref, b_ref, o_ref):
    dis = dis_ref[...]
    a = jnp.concatenate(
        [acc_ref[0, p] + acc_ref[1, p] + zs_ref[p] for p in range(2)],
        axis=-1) * dis
    h = jnp.dot(a, w_ref[...], preferred_element_type=jnp.float32) \
        + b_ref[...]
    o_ref[0] = jnp.maximum(h, 0.0) * dis


def _mm1_call(acc1, zs1, dis, w1, b1):
    return pl.pallas_call(
        _mm1_body,
        grid=(NBLK, 4),
        in_specs=[
            pl.BlockSpec((2, 2, BLK, 128), lambda i, q: (0, 0, i, 0)),
            pl.BlockSpec((2, BLK, 128), lambda i, q: (0, i, 0)),
            pl.BlockSpec((BLK, 1), lambda i, q: (i, 0)),
            pl.BlockSpec((D, 128), lambda i, q: (0, q)),
            pl.BlockSpec((1, 128), lambda i, q: (0, q)),
        ],
        out_specs=pl.BlockSpec((1, BLK, 128), lambda i, q: (q, i, 0)),
        out_shape=jax.ShapeDtypeStruct((4, N, 128), jnp.float32),
    )(acc1, zs1, dis, w1, b1.reshape(1, H))


def _mm2_body(acc_ref, zs_ref, dis_ref, w2_ref, b2_ref, u_ref, wacc_ref,
              o_ref):
    i = pl.program_id(0)
    dis = dis_ref[...]
    a = jnp.concatenate(
        [acc_ref[0, p] + acc_ref[1, p] + zs_ref[p] for p in range(4)],
        axis=-1) * dis
    h2 = jnp.dot(a, w2_ref[...], preferred_element_type=jnp.float32) \
        + b2_ref[...]
    h2 = jnp.maximum(h2, 0.0)
    rt = (u_ref[:, :G] + wacc_ref[0, :, :G] + wacc_ref[1, :, :G]) * dis
    contrib = lax.dot_general(rt, h2, (((0,), (0,)), ((), ())),
                              preferred_element_type=jnp.float32)

    @pl.when(i == 0)
    def _():
        o_ref[...] = jnp.zeros_like(o_ref)

    o_ref[...] += contrib


def _mm2_call(acc2, zs2, dis, w2, b2, u, wacc):
    return pl.pallas_call(
        _mm2_body,
        grid=(NBLK,),
        in_specs=[
            pl.BlockSpec((2, 4, BLK, 128), lambda i: (0, 0, i, 0)),
            pl.BlockSpec((4, BLK, 128), lambda i: (0, i, 0)),
            pl.BlockSpec((BLK, 1), lambda i: (i, 0)),
            pl.BlockSpec((H, H), lambda i: (0, 0)),
            pl.BlockSpec((1, H), lambda i: (0, 0)),
            pl.BlockSpec((BLK, 128), lambda i: (i, 0)),
            pl.BlockSpec((2, BLK, 128), lambda i: (0, i, 0)),
        ],
        out_specs=pl.BlockSpec((G, H), lambda i: (0, 0)),
        out_shape=jax.ShapeDtypeStruct((G, H), jnp.float32),
    )(acc2, zs2, dis, w2, b2.reshape(1, H), u, wacc)


def _final_body(pp_ref, w3_ref, b3_ref, cnt_ref, lw_ref, lb_ref, o_ref):
    pooled = jnp.dot(pp_ref[...], w3_ref[...],
                     preferred_element_type=jnp.float32)
    cp = jnp.transpose(
        (cnt_ref[...] > 0).astype(jnp.float32), (1, 0))
    pooled = pooled + b3_ref[...] * cp
    o_ref[...] = jnp.dot(pooled, lw_ref[...],
                         preferred_element_type=jnp.float32) + lb_ref[...]


def _final_call(pooled_pre, w3, b3, cnt, lin_w, lin_b):
    return pl.pallas_call(
        _final_body,
        out_shape=jax.ShapeDtypeStruct((G, C), jnp.float32),
    )(pooled_pre, w3, b3.reshape(1, H), cnt, lin_w, lin_b.reshape(1, C))


_sc_prop2 = _make_sc_prop(2, 128)
_sc_prop4 = _make_sc_prop(4, 128)
_sc_prop_rt = _make_sc_prop(1, 128)
_sc_degree = _make_sc_degree()


def kernel(x, edge_index, batch, bn_gamma, bn_beta, W1, b1, W2, b2, W3, b3,
           lin_W, lin_b):
    src = edge_index[0]
    dst = edge_index[1]
    padz = jnp.zeros((EPAD - E,), jnp.int32)
    padd = jnp.full((EPAD - E,), DUMMY, jnp.int32)
    g_src = jnp.concatenate([src, padz]).reshape(NW, NCH, CHUNK)
    s_dst = jnp.concatenate([dst, padd]).reshape(NW, NCH, CHUNK)
    g_dst = jnp.concatenate([dst, padz]).reshape(NW, NCH, CHUNK)
    s_src = jnp.concatenate([src, padd]).reshape(NW, NCH, CHUNK)

    deg2 = _sc_degree(s_dst)
    stats, cnt = _stats_call(x, batch)
    zs1, u, dis = _bn_call(x, bn_gamma, bn_beta, stats, cnt, deg2, batch)
    acc1 = _sc_prop2(zs1[0], zs1[1], g_src, s_dst)
    wacc = _sc_prop_rt(u, g_dst, s_src)
    zs2 = _mm1_call(acc1, zs1, dis, W1, b1)
    acc2 = _sc_prop4(zs2[0], zs2[1], zs2[2], zs2[3], g_src, s_dst)
    wacc1 = wacc[:, 0]
    pooled_pre = _mm2_call(acc2, zs2, dis, W2, b2, u, wacc1)
    return _final_call(pooled_pre, W3, b3, cnt, lin_W, lin_b)
